# baseline (speedup 1.0000x reference)
import functools

import jax
import jax.numpy as jnp
from jax import lax
from jax.experimental import pallas as pl
from jax.experimental.pallas import tpu as pltpu
from jax.experimental.pallas import tpu_sc as plsc

_N = 10000
_E = 320000
_D = 128
_R = 2 * _E
_NC, _NS = 2, 16
_CPC = _D // _NC

_CHUNK = 256
_NCHUNK = _R // _CHUNK
_SUB = 128
_KSUB = _CHUNK // _SUB
_ZROWS = _N // _NS

_GCHUNK = 64
_NGCHUNK = _R // _GCHUNK

_BLK = 8000
_HP = jax.lax.Precision.HIGHEST


def _sc_segsum_body(x_hbm, ei_hbm, z64_hbm, z16_hbm, ones_hbm,
                    s1_hbm, p_hbm, deg_hbm,
                    s1_sh, p_sh, deg_sh,
                    rows0, rows1, nidx0, nidx1, pidx0, pidx1,
                    ubuf, vbuf, ones_v, sem0, sem1):
    cid = lax.axis_index("c")
    sid = lax.axis_index("s")

    rs = sid * _ZROWS
    pltpu.sync_copy(z64_hbm, s1_sh.at[pl.ds(rs, _ZROWS)])
    pltpu.sync_copy(z64_hbm, p_sh.at[pl.ds(rs, _ZROWS)])
    pltpu.sync_copy(z16_hbm, deg_sh.at[pl.ds(rs, _ZROWS)])
    pltpu.sync_copy(ones_hbm, ones_v)
    plsc.subcore_barrier()

    cstart = cid * _CPC
    iot = lax.iota(jnp.int32, 16)

    slots = ((rows0, nidx0, pidx0, sem0), (rows1, nidx1, pidx1, sem1))

    def issue(k, slot):
        rows, idn, idp, sem = slots[slot]
        chunk = k * _NS + sid
        pltpu.async_copy(
            x_hbm.at[pl.ds(chunk * _CHUNK, _CHUNK), pl.ds(cstart, _CPC)],
            rows, sem)
        e0 = chunk * (_CHUNK // 2)
        pltpu.sync_copy(ei_hbm.at[0, pl.ds(e0, _CHUNK // 2)], ubuf)
        pltpu.sync_copy(ei_hbm.at[1, pl.ds(e0, _CHUNK // 2)], vbuf)
        for j in range(_KSUB):
            rj = iot * 0 + j
            for w in range(4):
                uu = ubuf[pl.ds(64 * j + 16 * w, 16)]
                vv = vbuf[pl.ds(64 * j + 16 * w, 16)]
                col = 32 * w + 2 * iot
                plsc.store_scatter(idn, [rj, col], uu)
                plsc.store_scatter(idn, [rj, col + 1], vv)
                plsc.store_scatter(idp, [rj, col], vv)
                plsc.store_scatter(idp, [rj, col + 1], uu)

    def drain(slot):
        rows, _, _, sem = slots[slot]
        pltpu.make_async_copy(
            x_hbm.at[pl.ds(0, _CHUNK), pl.ds(0, _CPC)], rows, sem).wait()

    def process(slot):
        rows, idn, idp, _ = slots[slot]
        for j in range(_KSUB):
            vals = rows.at[pl.ds(j * _SUB, _SUB), :]
            pltpu.sync_copy(vals, s1_sh.at[idn.at[j]], add=True)
            pltpu.sync_copy(vals, p_sh.at[idp.at[j]], add=True)

            @pl.when(cid == 0)
            def _():
                pltpu.sync_copy(ones_v, deg_sh.at[idn.at[j]], add=True)

    issue(0, 0)
    npair = (_NCHUNK // _NS + 2) // 2

    def step(k2, _):
        for s in (0, 1):
            kcur = k2 * 2 + s

            @pl.when((kcur + 1) * _NS + sid < _NCHUNK)
            def _():
                issue(kcur + 1, 1 - s)

            @pl.when(kcur * _NS + sid < _NCHUNK)
            def _():
                drain(s)
                process(s)
        return ()

    lax.fori_loop(0, npair, step, ())
    plsc.subcore_barrier()

    pltpu.sync_copy(s1_sh.at[pl.ds(rs, _ZROWS)],
                    s1_hbm.at[pl.ds(rs, _ZROWS), pl.ds(cstart, _CPC)])
    pltpu.sync_copy(p_sh.at[pl.ds(rs, _ZROWS)],
                    p_hbm.at[pl.ds(rs, _ZROWS), pl.ds(cstart, _CPC)])

    @pl.when(cid == 0)
    def _():
        pltpu.sync_copy(deg_sh.at[pl.ds(rs, _ZROWS)],
                        deg_hbm.at[pl.ds(rs, _ZROWS)])


def _sc_segsum(x, edge_index):
    z64 = jnp.zeros((_ZROWS, _CPC), jnp.float32)
    z16 = jnp.zeros((_ZROWS, 16), jnp.float32)
    ones = jnp.ones((_SUB, 16), jnp.float32)
    f = pl.kernel(
        _sc_segsum_body,
        out_type=[
            jax.ShapeDtypeStruct((_N, _D), jnp.float32),
            jax.ShapeDtypeStruct((_N, _D), jnp.float32),
            jax.ShapeDtypeStruct((_N, 16), jnp.float32),
        ],
        mesh=plsc.VectorSubcoreMesh(core_axis_name="c", subcore_axis_name="s"),
        scratch_types=[
            pltpu.VMEM_SHARED((_N, _CPC), jnp.float32),
            pltpu.VMEM_SHARED((_N, _CPC), jnp.float32),
            pltpu.VMEM_SHARED((_N, 16), jnp.float32),
            pltpu.VMEM((_CHUNK, _CPC), jnp.float32),
            pltpu.VMEM((_CHUNK, _CPC), jnp.float32),
            pltpu.VMEM((_KSUB, _SUB), jnp.int32),
            pltpu.VMEM((_KSUB, _SUB), jnp.int32),
            pltpu.VMEM((_KSUB, _SUB), jnp.int32),
            pltpu.VMEM((_KSUB, _SUB), jnp.int32),
            pltpu.VMEM((_CHUNK // 2,), jnp.int32),
            pltpu.VMEM((_CHUNK // 2,), jnp.int32),
            pltpu.VMEM((_SUB, 16), jnp.float32),
            pltpu.SemaphoreType.DMA,
            pltpu.SemaphoreType.DMA,
        ],
        compiler_params=pltpu.CompilerParams(use_tc_tiling_on_sc=False, needs_layout_passes=False),
    )
    return f(x, edge_index, z64, z16, ones)


def _tables_body(s1_ref, p_ref, deg_ref, w1_ref, ts_ref, tc_ref, w1a_ref,
                 st_ref):
    i = pl.program_id(0)
    s1 = s1_ref[...]
    p = p_ref[...]
    deg = deg_ref[:, 0:1]
    w11 = w1_ref[_D:2 * _D]
    w12 = w1_ref[2 * _D:3 * _D]
    w13 = w1_ref[3 * _D:4 * _D]
    w14 = w1_ref[4 * _D:5 * _D]
    w1a = w1_ref[0:_D] + w11
    ts = jnp.dot(s1, w11 + w12, precision=_HP,
                 preferred_element_type=jnp.float32)
    tc = (jnp.dot(deg * s1, w11 + w12 + w13 + w14, precision=_HP,
                  preferred_element_type=jnp.float32)
          + jnp.dot(p, w11 + w13, precision=_HP,
                    preferred_element_type=jnp.float32))
    ts_ref[...] = ts
    tc_ref[...] = tc
    w1a_ref[...] = w1a

    u = jnp.dot(s1, w1a, precision=_HP, preferred_element_type=jnp.float32)

    @pl.when(i == 0)
    def _():
        st_ref[...] = jnp.zeros_like(st_ref)

    st_ref[0:1, :] += jnp.sum(u + deg * (ts + tc), axis=0, keepdims=True)
    st_ref[1:2, :] += jnp.sum(deg * (ts * ts + tc * tc), axis=0,
                              keepdims=True)


def _tables(s1, p, deg16, w1):
    nb = 5
    blk = _N // nb
    return pl.pallas_call(
        _tables_body,
        grid=(nb,),
        in_specs=[
            pl.BlockSpec((blk, _D), lambda i: (i, 0)),
            pl.BlockSpec((blk, _D), lambda i: (i, 0)),
            pl.BlockSpec((blk, 16), lambda i: (i, 0)),
            pl.BlockSpec((5 * _D, 2 * _D), lambda i: (0, 0)),
        ],
        out_specs=[
            pl.BlockSpec((blk, 2 * _D), lambda i: (i, 0)),
            pl.BlockSpec((blk, 2 * _D), lambda i: (i, 0)),
            pl.BlockSpec((_D, 2 * _D), lambda i: (0, 0)),
            pl.BlockSpec((2, 2 * _D), lambda i: (0, 0)),
        ],
        out_shape=[
            jax.ShapeDtypeStruct((_N, 2 * _D), jnp.float32),
            jax.ShapeDtypeStruct((_N, 2 * _D), jnp.float32),
            jax.ShapeDtypeStruct((_D, 2 * _D), jnp.float32),
            jax.ShapeDtypeStruct((2, 2 * _D), jnp.float32),
        ],
        compiler_params=pltpu.CompilerParams(
            dimension_semantics=("arbitrary",)),
    )(s1, p, deg16, w1)


def _sc_gather_body(ts_hbm, tc_hbm, ei_hbm, g_hbm,
                    bufa0, bufb0, bufa1, bufb1,
                    nidx0, midx0, nidx1, midx1, ubuf, vbuf, sem0, sem1):
    cid = lax.axis_index("c")
    sid = lax.axis_index("s")
    wid = sid * _NC + cid
    nw = _NC * _NS
    iot = lax.iota(jnp.int32, 16)

    slots = ((bufa0, bufb0, nidx0, midx0, sem0),
             (bufa1, bufb1, nidx1, midx1, sem1))

    def issue(k, slot):
        a, b, idn, idm, sem = slots[slot]
        chunk = k * nw + wid
        e0 = chunk * (_GCHUNK // 2)
        pltpu.sync_copy(ei_hbm.at[0, pl.ds(e0, _GCHUNK // 2)], ubuf)
        pltpu.sync_copy(ei_hbm.at[1, pl.ds(e0, _GCHUNK // 2)], vbuf)
        for w in range(_GCHUNK // 32):
            uu = ubuf[pl.ds(16 * w, 16)]
            vv = vbuf[pl.ds(16 * w, 16)]
            col = 32 * w + 2 * iot
            plsc.store_scatter(idn, [col], uu)
            plsc.store_scatter(idn, [col + 1], vv)
            plsc.store_scatter(idm, [col], vv)
            plsc.store_scatter(idm, [col + 1], uu)
        pltpu.async_copy(ts_hbm.at[idm], a, sem)
        pltpu.async_copy(tc_hbm.at[idn], b, sem)

    def drain(slot):
        a, b, _, _, sem = slots[slot]
        dummy = ts_hbm.at[pl.ds(0, _GCHUNK), :]
        pltpu.make_async_copy(dummy, a, sem).wait()
        pltpu.make_async_copy(dummy, b, sem).wait()

    def process(k, slot):
        a, b, _, _, _ = slots[slot]

        def add_row(r, _):
            for cc in range(2 * _D // 16):
                sl = pl.ds(cc * 16, 16)
                a[r, sl] = a[r, sl] + b[r, sl]
            return ()

        lax.fori_loop(0, _GCHUNK, add_row, ())
        chunk = k * nw + wid
        pltpu.sync_copy(a, g_hbm.at[pl.ds(chunk * _GCHUNK, _GCHUNK), :])

    issue(0, 0)
    npair = (_NGCHUNK // nw + 2) // 2

    def step(k2, _):
        for s in (0, 1):
            kcur = k2 * 2 + s

            @pl.when((kcur + 1) * nw + wid < _NGCHUNK)
            def _():
                issue(kcur + 1, 1 - s)

            @pl.when(kcur * nw + wid < _NGCHUNK)
            def _():
                drain(s)
                process(kcur, s)
        return ()

    lax.fori_loop(0, npair, step, ())


def _sc_gather(ts, tc, edge_index):
    f = pl.kernel(
        _sc_gather_body,
        out_type=jax.ShapeDtypeStruct((_R, 2 * _D), jnp.float32),
        mesh=plsc.VectorSubcoreMesh(core_axis_name="c", subcore_axis_name="s"),
        scratch_types=[
            pltpu.VMEM((_GCHUNK, 2 * _D), jnp.float32),
            pltpu.VMEM((_GCHUNK, 2 * _D), jnp.float32),
            pltpu.VMEM((_GCHUNK, 2 * _D), jnp.float32),
            pltpu.VMEM((_GCHUNK, 2 * _D), jnp.float32),
            pltpu.VMEM((_GCHUNK,), jnp.int32),
            pltpu.VMEM((_GCHUNK,), jnp.int32),
            pltpu.VMEM((_GCHUNK,), jnp.int32),
            pltpu.VMEM((_GCHUNK,), jnp.int32),
            pltpu.VMEM((_GCHUNK // 2,), jnp.int32),
            pltpu.VMEM((_GCHUNK // 2,), jnp.int32),
            pltpu.SemaphoreType.DMA,
            pltpu.SemaphoreType.DMA,
        ],
        compiler_params=pltpu.CompilerParams(use_tc_tiling_on_sc=False, needs_layout_passes=False),
    )
    return f(ts, tc, edge_index)


def _l2_body(x_ref, g_ref, w1a_ref, sc_ref, sh_ref, w2_ref,
             t2_ref, stat_ref):
    i = pl.program_id(0)
    t = jnp.dot(x_ref[...], w1a_ref[...],
                preferred_element_type=jnp.float32
                ) + g_ref[...]
    h2 = jnp.maximum(t * sc_ref[...] + sh_ref[...], 0.0)
    t2 = jnp.dot(h2, w2_ref[...],
                 preferred_element_type=jnp.float32)
    t2_ref[...] = t2.astype(jnp.bfloat16)

    @pl.when(i == 0)
    def _():
        stat_ref[...] = jnp.zeros_like(stat_ref)

    stat_ref[0:1, :] += jnp.sum(t2, axis=0, keepdims=True)
    stat_ref[1:2, :] += jnp.sum(t2 * t2, axis=0, keepdims=True)


def _l2_pass(x, g, w1a, scale1, shift1, w2):
    nblk = _R // _BLK
    return pl.pallas_call(
        _l2_body,
        grid=(nblk,),
        in_specs=[
            pl.BlockSpec((_BLK, _D), lambda i: (i, 0)),
            pl.BlockSpec((_BLK, 2 * _D), lambda i: (i, 0)),
            pl.BlockSpec((_D, 2 * _D), lambda i: (0, 0)),
            pl.BlockSpec((1, 2 * _D), lambda i: (0, 0)),
            pl.BlockSpec((1, 2 * _D), lambda i: (0, 0)),
            pl.BlockSpec((2 * _D, _D), lambda i: (0, 0)),
        ],
        out_specs=[
            pl.BlockSpec((_BLK, _D), lambda i: (i, 0)),
            pl.BlockSpec((2, _D), lambda i: (0, 0)),
        ],
        out_shape=[
            jax.ShapeDtypeStruct((_R, _D), jnp.bfloat16),
            jax.ShapeDtypeStruct((2, _D), jnp.float32),
        ],
        compiler_params=pltpu.CompilerParams(
            dimension_semantics=("arbitrary",)),
    )(x, g, w1a, scale1, shift1, w2)


def _out_body(t2_ref, sc_ref, sh_ref, o_ref):
    o_ref[...] = jnp.maximum(
        t2_ref[...].astype(jnp.float32) * sc_ref[...] + sh_ref[...], 0.0)


def _out_pass(t2, scale2, shift2):
    nblk = _R // _BLK
    return pl.pallas_call(
        _out_body,
        grid=(nblk,),
        in_specs=[
            pl.BlockSpec((_BLK, _D), lambda i: (i, 0)),
            pl.BlockSpec((1, _D), lambda i: (0, 0)),
            pl.BlockSpec((1, _D), lambda i: (0, 0)),
        ],
        out_specs=pl.BlockSpec((_BLK, _D), lambda i: (i, 0)),
        out_shape=jax.ShapeDtypeStruct((_R, _D), jnp.float32),
        compiler_params=pltpu.CompilerParams(
            dimension_semantics=("arbitrary",)),
    )(t2, scale2, shift2)


def _bn_affine(stats, gamma, beta):
    mu = stats[0] / _R
    var = stats[1] / _R - mu * mu
    scale = gamma / jnp.sqrt(var + 1e-5)
    shift = beta - mu * scale
    return scale.reshape(1, -1), shift.reshape(1, -1)


def kernel(edge_rep, edge_index, W1, gamma1, beta1, W2, gamma2, beta2):
    s1, p, deg16 = _sc_segsum(edge_rep, edge_index)
    ts, tc, w1a, stats1 = _tables(s1, p, deg16, W1)
    g = _sc_gather(ts, tc, edge_index)
    scale1, shift1 = _bn_affine(stats1, gamma1, beta1)
    t2, stats2 = _l2_pass(edge_rep, g, w1a, scale1, shift1, W2)
    scale2, shift2 = _bn_affine(stats2, gamma2, beta2)
    return _out_pass(t2, scale2, shift2)


# speedup vs baseline: 1.1549x; 1.1549x over previous
import functools

import jax
import jax.numpy as jnp
from jax import lax
from jax.experimental import pallas as pl
from jax.experimental.pallas import tpu as pltpu
from jax.experimental.pallas import tpu_sc as plsc

_N = 10000
_E = 320000
_D = 128
_R = 2 * _E
_NC, _NS = 2, 16
_CPC = _D // _NC

_CHUNK = 256
_NCHUNK = _R // _CHUNK
_SUB = 128
_KSUB = _CHUNK // _SUB
_ZROWS = _N // _NS

_GCHUNK = 128
_NGCHUNK = _R // _GCHUNK

_BLK = 8000
_HP = jax.lax.Precision.HIGHEST


def _sc_segsum_body(x_hbm, ei_hbm, z64_hbm, z16_hbm, ones_hbm,
                    s1_hbm, p_hbm, deg_hbm,
                    s1_sh, p_sh, deg_sh,
                    rows0, rows1, nidx0, nidx1, pidx0, pidx1,
                    ubuf, vbuf, ones_v, sem0, sem1):
    cid = lax.axis_index("c")
    sid = lax.axis_index("s")

    rs = sid * _ZROWS
    pltpu.sync_copy(z64_hbm, s1_sh.at[pl.ds(rs, _ZROWS)])
    pltpu.sync_copy(z64_hbm, p_sh.at[pl.ds(rs, _ZROWS)])
    pltpu.sync_copy(z16_hbm, deg_sh.at[pl.ds(rs, _ZROWS)])
    pltpu.sync_copy(ones_hbm, ones_v)
    plsc.subcore_barrier()

    cstart = cid * _CPC
    iot = lax.iota(jnp.int32, 16)

    slots = ((rows0, nidx0, pidx0, sem0), (rows1, nidx1, pidx1, sem1))

    def issue(k, slot):
        rows, idn, idp, sem = slots[slot]
        chunk = k * _NS + sid
        pltpu.async_copy(
            x_hbm.at[pl.ds(chunk * _CHUNK, _CHUNK), pl.ds(cstart, _CPC)],
            rows, sem)
        e0 = chunk * (_CHUNK // 2)
        pltpu.sync_copy(ei_hbm.at[0, pl.ds(e0, _CHUNK // 2)], ubuf)
        pltpu.sync_copy(ei_hbm.at[1, pl.ds(e0, _CHUNK // 2)], vbuf)
        for j in range(_KSUB):
            rj = iot * 0 + j
            for w in range(4):
                uu = ubuf[pl.ds(64 * j + 16 * w, 16)]
                vv = vbuf[pl.ds(64 * j + 16 * w, 16)]
                col = 32 * w + 2 * iot
                plsc.store_scatter(idn, [rj, col], uu)
                plsc.store_scatter(idn, [rj, col + 1], vv)
                plsc.store_scatter(idp, [rj, col], vv)
                plsc.store_scatter(idp, [rj, col + 1], uu)

    def drain(slot):
        rows, _, _, sem = slots[slot]
        pltpu.make_async_copy(
            x_hbm.at[pl.ds(0, _CHUNK), pl.ds(0, _CPC)], rows, sem).wait()

    def process(slot):
        rows, idn, idp, _ = slots[slot]
        for j in range(_KSUB):
            vals = rows.at[pl.ds(j * _SUB, _SUB), :]
            pltpu.sync_copy(vals, s1_sh.at[idn.at[j]], add=True)
            pltpu.sync_copy(vals, p_sh.at[idp.at[j]], add=True)

            @pl.when(cid == 0)
            def _():
                pltpu.sync_copy(ones_v, deg_sh.at[idn.at[j]], add=True)

    issue(0, 0)
    npair = (_NCHUNK // _NS + 2) // 2

    def step(k2, _):
        for s in (0, 1):
            kcur = k2 * 2 + s

            @pl.when((kcur + 1) * _NS + sid < _NCHUNK)
            def _():
                issue(kcur + 1, 1 - s)

            @pl.when(kcur * _NS + sid < _NCHUNK)
            def _():
                drain(s)
                process(s)
        return ()

    lax.fori_loop(0, npair, step, ())
    plsc.subcore_barrier()

    pltpu.sync_copy(s1_sh.at[pl.ds(rs, _ZROWS)],
                    s1_hbm.at[pl.ds(rs, _ZROWS), pl.ds(cstart, _CPC)])
    pltpu.sync_copy(p_sh.at[pl.ds(rs, _ZROWS)],
                    p_hbm.at[pl.ds(rs, _ZROWS), pl.ds(cstart, _CPC)])

    @pl.when(cid == 0)
    def _():
        pltpu.sync_copy(deg_sh.at[pl.ds(rs, _ZROWS)],
                        deg_hbm.at[pl.ds(rs, _ZROWS)])


def _sc_segsum(x, edge_index):
    z64 = jnp.zeros((_ZROWS, _CPC), jnp.float32)
    z16 = jnp.zeros((_ZROWS, 16), jnp.float32)
    ones = jnp.ones((_SUB, 16), jnp.float32)
    f = pl.kernel(
        _sc_segsum_body,
        out_type=[
            jax.ShapeDtypeStruct((_N, _D), jnp.float32),
            jax.ShapeDtypeStruct((_N, _D), jnp.float32),
            jax.ShapeDtypeStruct((_N, 16), jnp.float32),
        ],
        mesh=plsc.VectorSubcoreMesh(core_axis_name="c", subcore_axis_name="s"),
        scratch_types=[
            pltpu.VMEM_SHARED((_N, _CPC), jnp.float32),
            pltpu.VMEM_SHARED((_N, _CPC), jnp.float32),
            pltpu.VMEM_SHARED((_N, 16), jnp.float32),
            pltpu.VMEM((_CHUNK, _CPC), jnp.float32),
            pltpu.VMEM((_CHUNK, _CPC), jnp.float32),
            pltpu.VMEM((_KSUB, _SUB), jnp.int32),
            pltpu.VMEM((_KSUB, _SUB), jnp.int32),
            pltpu.VMEM((_KSUB, _SUB), jnp.int32),
            pltpu.VMEM((_KSUB, _SUB), jnp.int32),
            pltpu.VMEM((_CHUNK // 2,), jnp.int32),
            pltpu.VMEM((_CHUNK // 2,), jnp.int32),
            pltpu.VMEM((_SUB, 16), jnp.float32),
            pltpu.SemaphoreType.DMA,
            pltpu.SemaphoreType.DMA,
        ],
        compiler_params=pltpu.CompilerParams(use_tc_tiling_on_sc=False, needs_layout_passes=False),
    )
    return f(x, edge_index, z64, z16, ones)


def _tables_body(s1_ref, p_ref, deg_ref, w1_ref, ts_ref, tc_ref, w1a_ref,
                 st_ref):
    i = pl.program_id(0)
    s1 = s1_ref[...]
    p = p_ref[...]
    deg = deg_ref[:, 0:1]
    w11 = w1_ref[_D:2 * _D]
    w12 = w1_ref[2 * _D:3 * _D]
    w13 = w1_ref[3 * _D:4 * _D]
    w14 = w1_ref[4 * _D:5 * _D]
    w1a = w1_ref[0:_D] + w11
    ts = jnp.dot(s1, w11 + w12, precision=_HP,
                 preferred_element_type=jnp.float32)
    tc = (jnp.dot(deg * s1, w11 + w12 + w13 + w14, precision=_HP,
                  preferred_element_type=jnp.float32)
          + jnp.dot(p, w11 + w13, precision=_HP,
                    preferred_element_type=jnp.float32))
    ts_ref[...] = ts.astype(jnp.bfloat16)
    tc_ref[...] = tc.astype(jnp.bfloat16)
    w1a_ref[...] = w1a

    u = jnp.dot(s1, w1a, precision=_HP, preferred_element_type=jnp.float32)

    @pl.when(i == 0)
    def _():
        st_ref[...] = jnp.zeros_like(st_ref)

    st_ref[0:1, :] += jnp.sum(u + deg * (ts + tc), axis=0, keepdims=True)
    st_ref[1:2, :] += jnp.sum(deg * (ts * ts + tc * tc), axis=0,
                              keepdims=True)


def _tables(s1, p, deg16, w1):
    nb = 5
    blk = _N // nb
    return pl.pallas_call(
        _tables_body,
        grid=(nb,),
        in_specs=[
            pl.BlockSpec((blk, _D), lambda i: (i, 0)),
            pl.BlockSpec((blk, _D), lambda i: (i, 0)),
            pl.BlockSpec((blk, 16), lambda i: (i, 0)),
            pl.BlockSpec((5 * _D, 2 * _D), lambda i: (0, 0)),
        ],
        out_specs=[
            pl.BlockSpec((blk, 2 * _D), lambda i: (i, 0)),
            pl.BlockSpec((blk, 2 * _D), lambda i: (i, 0)),
            pl.BlockSpec((_D, 2 * _D), lambda i: (0, 0)),
            pl.BlockSpec((2, 2 * _D), lambda i: (0, 0)),
        ],
        out_shape=[
            jax.ShapeDtypeStruct((_N, 2 * _D), jnp.bfloat16),
            jax.ShapeDtypeStruct((_N, 2 * _D), jnp.bfloat16),
            jax.ShapeDtypeStruct((_D, 2 * _D), jnp.float32),
            jax.ShapeDtypeStruct((2, 2 * _D), jnp.float32),
        ],
        compiler_params=pltpu.CompilerParams(
            dimension_semantics=("arbitrary",)),
    )(s1, p, deg16, w1)


def _sc_gather_body(ts_hbm, tc_hbm, ei_hbm, g_hbm,
                    bufa0, bufb0, bufa1, bufb1,
                    nidx0, midx0, nidx1, midx1, ubuf, vbuf, sem0, sem1):
    cid = lax.axis_index("c")
    sid = lax.axis_index("s")
    wid = sid * _NC + cid
    nw = _NC * _NS
    iot = lax.iota(jnp.int32, 16)

    slots = ((bufa0, bufb0, nidx0, midx0, sem0),
             (bufa1, bufb1, nidx1, midx1, sem1))

    def issue(k, slot):
        a, b, idn, idm, sem = slots[slot]
        chunk = k * nw + wid
        e0 = chunk * (_GCHUNK // 2)
        pltpu.sync_copy(ei_hbm.at[0, pl.ds(e0, _GCHUNK // 2)], ubuf)
        pltpu.sync_copy(ei_hbm.at[1, pl.ds(e0, _GCHUNK // 2)], vbuf)
        for w in range(4):
            uu = ubuf[pl.ds(16 * w, 16)]
            vv = vbuf[pl.ds(16 * w, 16)]
            col = 32 * w + 2 * iot
            plsc.store_scatter(idn, [col], uu)
            plsc.store_scatter(idn, [col + 1], vv)
            plsc.store_scatter(idm, [col], vv)
            plsc.store_scatter(idm, [col + 1], uu)
        pltpu.async_copy(ts_hbm.at[idm], a, sem)
        pltpu.async_copy(tc_hbm.at[idn], b, sem)

    def drain(slot):
        a, b, _, _, sem = slots[slot]
        dummy = ts_hbm.at[pl.ds(0, _GCHUNK), :]
        pltpu.make_async_copy(dummy, a, sem).wait()
        pltpu.make_async_copy(dummy, b, sem).wait()

    def process(k, slot):
        a, b, _, _, _ = slots[slot]

        def add_row(r, _):
            for cc in range(2 * _D // 32):
                sl = pl.ds(cc * 32, 32)
                a[r, sl] = a[r, sl] + b[r, sl]
            return ()

        lax.fori_loop(0, _GCHUNK, add_row, ())
        chunk = k * nw + wid
        pltpu.sync_copy(a, g_hbm.at[pl.ds(chunk * _GCHUNK, _GCHUNK), :])

    issue(0, 0)
    npair = (_NGCHUNK // nw + 2) // 2

    def step(k2, _):
        for s in (0, 1):
            kcur = k2 * 2 + s

            @pl.when((kcur + 1) * nw + wid < _NGCHUNK)
            def _():
                issue(kcur + 1, 1 - s)

            @pl.when(kcur * nw + wid < _NGCHUNK)
            def _():
                drain(s)
                process(kcur, s)
        return ()

    lax.fori_loop(0, npair, step, ())


def _sc_gather(ts, tc, edge_index):
    f = pl.kernel(
        _sc_gather_body,
        out_type=jax.ShapeDtypeStruct((_R, 2 * _D), jnp.bfloat16),
        mesh=plsc.VectorSubcoreMesh(core_axis_name="c", subcore_axis_name="s"),
        scratch_types=[
            pltpu.VMEM((_GCHUNK, 2 * _D), jnp.bfloat16),
            pltpu.VMEM((_GCHUNK, 2 * _D), jnp.bfloat16),
            pltpu.VMEM((_GCHUNK, 2 * _D), jnp.bfloat16),
            pltpu.VMEM((_GCHUNK, 2 * _D), jnp.bfloat16),
            pltpu.VMEM((_GCHUNK,), jnp.int32),
            pltpu.VMEM((_GCHUNK,), jnp.int32),
            pltpu.VMEM((_GCHUNK,), jnp.int32),
            pltpu.VMEM((_GCHUNK,), jnp.int32),
            pltpu.VMEM((_GCHUNK // 2,), jnp.int32),
            pltpu.VMEM((_GCHUNK // 2,), jnp.int32),
            pltpu.SemaphoreType.DMA,
            pltpu.SemaphoreType.DMA,
        ],
        compiler_params=pltpu.CompilerParams(use_tc_tiling_on_sc=False, needs_layout_passes=False),
    )
    return f(ts, tc, edge_index)


def _l2_body(x_ref, g_ref, w1a_ref, sc_ref, sh_ref, w2_ref,
             t2_ref, stat_ref):
    i = pl.program_id(0)
    t = jnp.dot(x_ref[...], w1a_ref[...],
                preferred_element_type=jnp.float32
                ) + g_ref[...].astype(jnp.float32)
    h2 = jnp.maximum(t * sc_ref[...] + sh_ref[...], 0.0)
    t2 = jnp.dot(h2, w2_ref[...],
                 preferred_element_type=jnp.float32)
    t2_ref[...] = t2.astype(jnp.bfloat16)

    @pl.when(i == 0)
    def _():
        stat_ref[...] = jnp.zeros_like(stat_ref)

    stat_ref[0:1, :] += jnp.sum(t2, axis=0, keepdims=True)
    stat_ref[1:2, :] += jnp.sum(t2 * t2, axis=0, keepdims=True)


def _l2_pass(x, g, w1a, scale1, shift1, w2):
    nblk = _R // _BLK
    return pl.pallas_call(
        _l2_body,
        grid=(nblk,),
        in_specs=[
            pl.BlockSpec((_BLK, _D), lambda i: (i, 0)),
            pl.BlockSpec((_BLK, 2 * _D), lambda i: (i, 0)),
            pl.BlockSpec((_D, 2 * _D), lambda i: (0, 0)),
            pl.BlockSpec((1, 2 * _D), lambda i: (0, 0)),
            pl.BlockSpec((1, 2 * _D), lambda i: (0, 0)),
            pl.BlockSpec((2 * _D, _D), lambda i: (0, 0)),
        ],
        out_specs=[
            pl.BlockSpec((_BLK, _D), lambda i: (i, 0)),
            pl.BlockSpec((2, _D), lambda i: (0, 0)),
        ],
        out_shape=[
            jax.ShapeDtypeStruct((_R, _D), jnp.bfloat16),
            jax.ShapeDtypeStruct((2, _D), jnp.float32),
        ],
        compiler_params=pltpu.CompilerParams(
            dimension_semantics=("arbitrary",)),
    )(x, g, w1a, scale1, shift1, w2)


def _out_body(t2_ref, sc_ref, sh_ref, o_ref):
    o_ref[...] = jnp.maximum(
        t2_ref[...].astype(jnp.float32) * sc_ref[...] + sh_ref[...], 0.0)


def _out_pass(t2, scale2, shift2):
    nblk = _R // _BLK
    return pl.pallas_call(
        _out_body,
        grid=(nblk,),
        in_specs=[
            pl.BlockSpec((_BLK, _D), lambda i: (i, 0)),
            pl.BlockSpec((1, _D), lambda i: (0, 0)),
            pl.BlockSpec((1, _D), lambda i: (0, 0)),
        ],
        out_specs=pl.BlockSpec((_BLK, _D), lambda i: (i, 0)),
        out_shape=jax.ShapeDtypeStruct((_R, _D), jnp.float32),
        compiler_params=pltpu.CompilerParams(
            dimension_semantics=("arbitrary",)),
    )(t2, scale2, shift2)


def _bn_affine(stats, gamma, beta):
    mu = stats[0] / _R
    var = stats[1] / _R - mu * mu
    scale = gamma / jnp.sqrt(var + 1e-5)
    shift = beta - mu * scale
    return scale.reshape(1, -1), shift.reshape(1, -1)


def kernel(edge_rep, edge_index, W1, gamma1, beta1, W2, gamma2, beta2):
    s1, p, deg16 = _sc_segsum(edge_rep, edge_index)
    ts, tc, w1a, stats1 = _tables(s1, p, deg16, W1)
    g = _sc_gather(ts, tc, edge_index)
    scale1, shift1 = _bn_affine(stats1, gamma1, beta1)
    t2, stats2 = _l2_pass(edge_rep, g, w1a, scale1, shift1, W2)
    scale2, shift2 = _bn_affine(stats2, gamma2, beta2)
    return _out_pass(t2, scale2, shift2)
